# baseline (speedup 1.0000x reference)
import functools

import jax
import jax.numpy as jnp
from jax import lax
from jax.experimental import pallas as pl
from jax.experimental.pallas import tpu as pltpu
from jax.experimental.pallas import tpu_sc as plsc

_L = 16
_NT = 16


def _lane_bcast(v, lane):
    idx = jnp.full((_L,), lane, dtype=jnp.int32)
    dnums = lax.GatherDimensionNumbers(
        offset_dims=(), collapsed_slice_dims=(0,), start_index_map=(0,))
    return lax.gather(v, idx[:, None], dnums, slice_sizes=(1,),
                      mode=lax.GatherScatterMode.PROMISE_IN_BOUNDS)


def _lane_allsum(v):
    iota = lax.iota(jnp.int32, _L)
    dnums = lax.GatherDimensionNumbers(
        offset_dims=(), collapsed_slice_dims=(0,), start_index_map=(0,))
    for sh in (1, 2, 4, 8):
        idx = lax.bitwise_xor(iota, jnp.int32(sh))
        v = v + lax.gather(v, idx[:, None], dnums, slice_sizes=(1,),
                           mode=lax.GatherScatterMode.PROMISE_IN_BOUNDS)
    return v


def _step(g, p, acc):
    accd, s1g, s1p, s2g, s2p = acc
    d = g - p
    return (accd + d * d, s1g + g, s1p + p, s2g + g * g, s2p + p * p)


def _accumulate(gt_v, pred_v, n_elems):
    z = jnp.zeros((_L,), jnp.int32)
    acc = (z, z, z, z, z)
    n_full = n_elems // _L
    rem = n_elems - n_full * _L
    for j in range(n_full):
        acc = _step(gt_v[pl.ds(j * _L, _L)], pred_v[pl.ds(j * _L, _L)], acc)
    if rem:
        mask = lax.iota(jnp.int32, _L) >= jnp.int32(_L - rem)
        g = jnp.where(mask, gt_v[pl.ds(n_elems - _L, _L)], 0)
        p = jnp.where(mask, pred_v[pl.ds(n_elems - _L, _L)], 0)
        acc = _step(g, p, acc)
    return acc


def _pack_partials(acc):
    iota = lax.iota(jnp.int32, _L)
    packed = jnp.zeros((_L,), jnp.int32)
    for lane, v in enumerate(acc):
        packed = jnp.where(iota == lane, _lane_allsum(v), packed)
    return packed


@functools.cache
def _kappa_sc(n: int):
    chunk = ((n + _NT - 1) // _NT + _L - 1) // _L * _L
    last = n - (_NT - 1) * chunk
    mesh = plsc.VectorSubcoreMesh(
        core_axis_name="c", subcore_axis_name="s", num_cores=1)

    @functools.partial(
        pl.kernel,
        mesh=mesh,
        out_type=jax.ShapeDtypeStruct((1,), jnp.float32),
        scratch_types=[
            pltpu.VMEM((chunk,), jnp.int32),
            pltpu.VMEM((chunk,), jnp.int32),
            pltpu.VMEM((_L,), jnp.int32),
            pltpu.VMEM((_NT * _L,), jnp.int32),
            pltpu.VMEM((_L,), jnp.float32),
            pltpu.VMEM_SHARED((_NT * _L,), jnp.int32),
            pltpu.SemaphoreType.DMA,
            pltpu.SemaphoreType.DMA,
        ],
    )
    def kern(pred_hbm, gt_hbm, out_hbm, pred_v, gt_v, part_v, red_v, out_v,
             shared, sem_p, sem_g):
        wid = lax.axis_index("s")
        base = wid * chunk

        @pl.when(wid < _NT - 1)
        def _():
            cp_p = pltpu.async_copy(
                pred_hbm.at[pl.ds(base, chunk)], pred_v, sem_p)
            cp_g = pltpu.async_copy(
                gt_hbm.at[pl.ds(base, chunk)], gt_v, sem_g)
            cp_p.wait()
            cp_g.wait()
            part_v[...] = _pack_partials(_accumulate(gt_v, pred_v, chunk))

        @pl.when(wid == _NT - 1)
        def _():
            cp_p = pltpu.async_copy(
                pred_hbm.at[pl.ds((_NT - 1) * chunk, last)],
                pred_v.at[pl.ds(0, last)], sem_p)
            cp_g = pltpu.async_copy(
                gt_hbm.at[pl.ds((_NT - 1) * chunk, last)],
                gt_v.at[pl.ds(0, last)], sem_g)
            cp_p.wait()
            cp_g.wait()
            part_v[...] = _pack_partials(_accumulate(gt_v, pred_v, last))

        pltpu.sync_copy(part_v, shared.at[pl.ds(wid * _L, _L)])
        plsc.subcore_barrier()

        @pl.when(wid == 0)
        def _():
            pltpu.sync_copy(shared, red_v)
            tot = red_v[pl.ds(0, _L)]
            for s in range(1, _NT):
                tot = tot + red_v[pl.ds(s * _L, _L)]
            totf = tot.astype(jnp.float32)
            vd = _lane_bcast(totf, 0)
            v1g = _lane_bcast(totf, 1)
            v1p = _lane_bcast(totf, 2)
            v2g = _lane_bcast(totf, 3)
            v2p = _lane_bcast(totf, 4)
            nf = jnp.float32(n)
            den = nf * (v2g + v2p) - 2.0 * v1g * v1p
            res = 1.0 - nf * vd / den
            out_v[...] = res
            pltpu.sync_copy(out_v.at[pl.ds(0, 1)], out_hbm)

    return kern


def kernel(y_pred, y_gt):
    y_pred = jnp.ravel(y_pred).astype(jnp.int32)
    y_gt = jnp.ravel(y_gt).astype(jnp.int32)
    n = y_gt.shape[0]
    out = _kappa_sc(n)(y_pred, y_gt)
    return jnp.reshape(out, ())
